# baseline (speedup 1.0000x reference)
import functools

import jax
import jax.numpy as jnp
from jax import lax
from jax.experimental import pallas as pl
from jax.experimental.pallas import tpu as pltpu
from jax.experimental.pallas import tpu_sc as plsc

N = 16384
D = 64
T = 8192
MARGIN = 0.2
L = 16
W = 128


def _build_loss(num_cores, num_subcores):
    NW = num_cores * num_subcores
    TPW = T // NW
    IDX_ROWS = TPW // 128

    mesh = plsc.VectorSubcoreMesh(core_axis_name="c", subcore_axis_name="s")

    @functools.partial(
        pl.kernel,
        mesh=mesh,
        compiler_params=pltpu.CompilerParams(needs_layout_passes=False),
        out_type=jax.ShapeDtypeStruct((num_cores, W), jnp.float32),
        scratch_types=[
            pltpu.VMEM((IDX_ROWS, 128), jnp.int32),
            pltpu.VMEM((IDX_ROWS, 128), jnp.int32),
            pltpu.VMEM((IDX_ROWS, 128), jnp.int32),
            pltpu.VMEM((TPW,), jnp.int32),
            pltpu.VMEM((TPW,), jnp.int32),
            pltpu.VMEM((TPW,), jnp.int32),
            pltpu.VMEM((TPW, W), jnp.float32),
            pltpu.VMEM((TPW, W), jnp.float32),
            pltpu.VMEM((TPW, W), jnp.float32),
            pltpu.VMEM((TPW * L,), jnp.float32),
            pltpu.VMEM((W,), jnp.float32),
            pltpu.VMEM_SHARED((num_subcores, W), jnp.float32),
            pltpu.VMEM((num_subcores, W), jnp.float32),
            pltpu.SemaphoreType.DMA,
        ],
    )
    def triplet_loss_kernel(eeg_hbm, img_hbm, arow_hbm, prow_hbm, nrow_hbm,
                            aoff_hbm, poff_hbm, noff_hbm, out_hbm,
                            arow_v, prow_v, nrow_v, aoff_v, poff_v, noff_v,
                            a_v, p_v, n_v, s_v, part_v, shared, red_v, sem):
        cid = lax.axis_index("c")
        sid = lax.axis_index("s")
        wid = cid * num_subcores + sid

        idx_rows = pl.ds(wid * IDX_ROWS, IDX_ROWS)
        offs = pl.ds(wid * TPW, TPW)
        staged = (
            pltpu.async_copy(arow_hbm.at[idx_rows], arow_v, sem),
            pltpu.async_copy(prow_hbm.at[idx_rows], prow_v, sem),
            pltpu.async_copy(nrow_hbm.at[idx_rows], nrow_v, sem),
            pltpu.async_copy(aoff_hbm.at[offs], aoff_v, sem),
            pltpu.async_copy(poff_hbm.at[offs], poff_v, sem),
            pltpu.async_copy(noff_hbm.at[offs], noff_v, sem),
        )
        for cp in staged:
            cp.wait()

        copies = []
        for c in range(IDX_ROWS):
            dst = pl.ds(c * 128, 128)
            copies.append((pltpu.async_copy(eeg_hbm.at[arow_v.at[c]],
                                            a_v.at[dst], sem),
                           pltpu.async_copy(img_hbm.at[prow_v.at[c]],
                                            p_v.at[dst], sem),
                           pltpu.async_copy(img_hbm.at[nrow_v.at[c]],
                                            n_v.at[dst], sem)))

        GPC = 128 // L

        def pass1(g):
            base = g * L
            ao = aoff_v[pl.ds(base, L)]
            po = poff_v[pl.ds(base, L)]
            no = noff_v[pl.ds(base, L)]
            for j in range(L):
                i = base + j
                ba, bp, bn = ao[j], po[j], no[j]
                s = jnp.zeros((L,), jnp.float32)
                for k in range(D // L):
                    s = (s + a_v[i, pl.ds(ba + k * L, L)]
                         * (n_v[i, pl.ds(bn + k * L, L)]
                            - p_v[i, pl.ds(bp + k * L, L)]))
                s_v[pl.ds(i * L, L)] = s

        for c in range(IDX_ROWS):
            for cp in copies[c]:
                cp.wait()
            plsc.parallel_loop(c * GPC, (c + 1) * GPC)(pass1)

        def pass2(g, acc):
            rows = g * (L * L) + lax.iota(jnp.int32, L) * L
            t = jnp.zeros((L,), jnp.float32)
            for j in range(L):
                t = t + plsc.load_gather(s_v, [rows + j])
            return acc + jnp.maximum(t + MARGIN, 0.0)

        acc = plsc.parallel_loop(0, TPW // L, unroll=2,
                                 carry=jnp.zeros((L,), jnp.float32))(pass2)

        part_v[pl.ds(0, L)] = acc
        pltpu.sync_copy(part_v, shared.at[sid])
        plsc.subcore_barrier()

        @pl.when(sid == 0)
        def _():
            pltpu.sync_copy(shared, red_v)
            tot = jnp.zeros((L,), jnp.float32)
            for r in range(num_subcores):
                tot = tot + red_v[r, pl.ds(0, L)]
            total = jnp.sum(tot)
            lane = lax.iota(jnp.int32, L)
            part_v[pl.ds(0, L)] = jnp.where(lane == 0, total, 0.0)
            pltpu.sync_copy(part_v, out_hbm.at[cid])

    return triplet_loss_kernel


def kernel(eeg_embeddings, img_embeddings, target, triplets):
    info = plsc.get_sparse_core_info()
    num_cores, num_subcores = info.num_cores, info.num_subcores
    half = N // 2
    eeg_p = jnp.concatenate(
        [eeg_embeddings[:half], eeg_embeddings[half:]], axis=1)
    img_p = jnp.concatenate(
        [img_embeddings[:half], img_embeddings[half:]], axis=1)
    tri = triplets.astype(jnp.int32)
    rows = tri & (half - 1)
    offs = (tri >> 13) * D
    arow = rows[:, 0].reshape(T // 128, 128)
    prow = rows[:, 1].reshape(T // 128, 128)
    nrow = rows[:, 2].reshape(T // 128, 128)
    out = _build_loss(num_cores, num_subcores)(
        eeg_p, img_p, arow, prow, nrow,
        offs[:, 0], offs[:, 1], offs[:, 2])
    loss = jnp.sum(out[:, 0]) * (1.0 / T)
    return (loss, jnp.asarray(T))


# speedup vs baseline: 1.1954x; 1.1954x over previous
import functools

import jax
import jax.numpy as jnp
from jax import lax
from jax.experimental import pallas as pl
from jax.experimental.pallas import tpu as pltpu
from jax.experimental.pallas import tpu_sc as plsc

N = 16384
D = 64
T = 8192
MARGIN = 0.2
L = 16


def _build_kernel(num_cores, num_subcores):
    NW = num_cores * num_subcores
    TPW = T // NW
    CH = 64
    IDX_ROWS = TPW // CH

    mesh = plsc.VectorSubcoreMesh(core_axis_name="c", subcore_axis_name="s")

    @functools.partial(
        pl.kernel,
        mesh=mesh,
        compiler_params=pltpu.CompilerParams(needs_layout_passes=False,
                                             use_tc_tiling_on_sc=False),
        out_type=jax.ShapeDtypeStruct((num_cores, L), jnp.float32),
        scratch_types=[
            pltpu.VMEM((IDX_ROWS, CH), jnp.int32),
            pltpu.VMEM((IDX_ROWS, CH), jnp.int32),
            pltpu.VMEM((IDX_ROWS, CH), jnp.int32),
            pltpu.VMEM((TPW, D), jnp.float32),
            pltpu.VMEM((TPW, D), jnp.float32),
            pltpu.VMEM((TPW, D), jnp.float32),
            pltpu.VMEM((TPW * L,), jnp.float32),
            pltpu.VMEM((L,), jnp.float32),
            pltpu.VMEM_SHARED((num_subcores, L), jnp.float32),
            pltpu.VMEM((num_subcores, L), jnp.float32),
            pltpu.SemaphoreType.DMA,
        ],
    )
    def triplet_loss_kernel(eeg_hbm, img_hbm, aidx_hbm, pidx_hbm, nidx_hbm,
                            out_hbm, aidx_v, pidx_v, nidx_v, a_v, p_v, n_v,
                            s_v, part_v, shared, red_v, sem):
        cid = lax.axis_index("c")
        sid = lax.axis_index("s")
        wid = cid * num_subcores + sid

        i0 = pltpu.async_copy(aidx_hbm.at[pl.ds(wid * IDX_ROWS, IDX_ROWS)],
                              aidx_v, sem)
        i1 = pltpu.async_copy(pidx_hbm.at[pl.ds(wid * IDX_ROWS, IDX_ROWS)],
                              pidx_v, sem)
        i2 = pltpu.async_copy(nidx_hbm.at[pl.ds(wid * IDX_ROWS, IDX_ROWS)],
                              nidx_v, sem)
        i0.wait()
        i1.wait()
        i2.wait()

        copies = []
        for c in range(IDX_ROWS):
            dst = pl.ds(c * CH, CH)
            copies.append((pltpu.async_copy(eeg_hbm.at[aidx_v.at[c]],
                                            a_v.at[dst], sem),
                           pltpu.async_copy(img_hbm.at[pidx_v.at[c]],
                                            p_v.at[dst], sem),
                           pltpu.async_copy(img_hbm.at[nidx_v.at[c]],
                                            n_v.at[dst], sem)))

        def pass1(i, _):
            s = jnp.zeros((L,), jnp.float32)
            for k in range(D // L):
                ck = pl.ds(k * L, L)
                s = s + a_v[i, ck] * (n_v[i, ck] - p_v[i, ck])
            s_v[pl.ds(i * L, L)] = s

        for c in range(IDX_ROWS):
            for cp in copies[c]:
                cp.wait()
            plsc.parallel_loop(c * CH, (c + 1) * CH, unroll=4)(
                lambda i: pass1(i, None))

        def pass2(g, acc):
            rows = g * (L * L) + lax.iota(jnp.int32, L) * L
            t = jnp.zeros((L,), jnp.float32)
            for j in range(L):
                t = t + plsc.load_gather(s_v, [rows + j])
            return acc + jnp.maximum(t + MARGIN, 0.0)

        acc = plsc.parallel_loop(0, TPW // L, unroll=2,
                                 carry=jnp.zeros((L,), jnp.float32))(pass2)

        part_v[:] = acc
        pltpu.sync_copy(part_v, shared.at[sid])
        plsc.subcore_barrier()

        @pl.when(sid == 0)
        def _():
            pltpu.sync_copy(shared, red_v)
            tot = jnp.zeros((L,), jnp.float32)
            for r in range(num_subcores):
                tot = tot + red_v[r, :]
            total = jnp.sum(tot)
            lane = lax.iota(jnp.int32, L)
            part_v[:] = jnp.where(lane == 0, total, 0.0)
            pltpu.sync_copy(part_v, out_hbm.at[cid])

    return triplet_loss_kernel


def kernel(eeg_embeddings, img_embeddings, target, triplets):
    info = plsc.get_sparse_core_info()
    num_cores, num_subcores = info.num_cores, info.num_subcores
    tri = triplets.astype(jnp.int32)
    aidx = tri[:, 0].reshape(T // 64, 64)
    pidx = tri[:, 1].reshape(T // 64, 64)
    nidx = tri[:, 2].reshape(T // 64, 64)
    fn = _build_kernel(num_cores, num_subcores)
    out = fn(eeg_embeddings, img_embeddings, aidx, pidx, nidx)
    loss = jnp.sum(out[:, 0]) * (1.0 / T)
    return (loss, jnp.asarray(T))


# speedup vs baseline: 1.2121x; 1.0140x over previous
import functools

import jax
import jax.numpy as jnp
from jax import lax
from jax.experimental import pallas as pl
from jax.experimental.pallas import tpu as pltpu
from jax.experimental.pallas import tpu_sc as plsc

N = 16384
D = 64
T = 8192
MARGIN = 0.2
L = 16


def _build_kernel(num_cores, num_subcores):
    NW = num_cores * num_subcores
    TPW = T // NW
    CH = 64
    IDX_ROWS = TPW // CH

    mesh = plsc.VectorSubcoreMesh(core_axis_name="c", subcore_axis_name="s")

    @functools.partial(
        pl.kernel,
        mesh=mesh,
        compiler_params=pltpu.CompilerParams(needs_layout_passes=False,
                                             use_tc_tiling_on_sc=False),
        out_type=jax.ShapeDtypeStruct((num_cores, L), jnp.float32),
        scratch_types=[
            pltpu.VMEM((IDX_ROWS, CH), jnp.int32),
            pltpu.VMEM((IDX_ROWS, CH), jnp.int32),
            pltpu.VMEM((IDX_ROWS, CH), jnp.int32),
            pltpu.VMEM((TPW, D), jnp.float32),
            pltpu.VMEM((TPW, D), jnp.float32),
            pltpu.VMEM((TPW, D), jnp.float32),
            pltpu.VMEM((L,), jnp.float32),
            pltpu.VMEM_SHARED((num_subcores, L), jnp.float32),
            pltpu.VMEM((num_subcores, L), jnp.float32),
            pltpu.SemaphoreType.DMA,
        ],
    )
    def triplet_loss_kernel(eeg_hbm, img_hbm, aidx_hbm, pidx_hbm, nidx_hbm,
                            out_hbm, aidx_v, pidx_v, nidx_v, a_v, p_v, n_v,
                            part_v, shared, red_v, sem):
        cid = lax.axis_index("c")
        sid = lax.axis_index("s")
        wid = cid * num_subcores + sid

        i0 = pltpu.async_copy(aidx_hbm.at[pl.ds(wid * IDX_ROWS, IDX_ROWS)],
                              aidx_v, sem)
        i1 = pltpu.async_copy(pidx_hbm.at[pl.ds(wid * IDX_ROWS, IDX_ROWS)],
                              pidx_v, sem)
        i2 = pltpu.async_copy(nidx_hbm.at[pl.ds(wid * IDX_ROWS, IDX_ROWS)],
                              nidx_v, sem)
        i0.wait()
        i1.wait()
        i2.wait()

        copies = []
        for c in range(IDX_ROWS):
            dst = pl.ds(c * CH, CH)
            copies.append((pltpu.async_copy(eeg_hbm.at[aidx_v.at[c]],
                                            a_v.at[dst], sem),
                           pltpu.async_copy(img_hbm.at[pidx_v.at[c]],
                                            p_v.at[dst], sem),
                           pltpu.async_copy(img_hbm.at[nidx_v.at[c]],
                                            n_v.at[dst], sem)))

        def pass1(i, acc):
            s = jnp.zeros((L,), jnp.float32)
            for k in range(D // L):
                ck = pl.ds(k * L, L)
                s = s + a_v[i, ck] * (n_v[i, ck] - p_v[i, ck])
            return acc + jnp.maximum(jnp.sum(s) + MARGIN, 0.0)

        acc = jnp.float32(0.0)
        for c in range(IDX_ROWS):
            for cp in copies[c]:
                cp.wait()
            acc = plsc.parallel_loop(c * CH, (c + 1) * CH, unroll=4,
                                     carry=acc)(pass1)

        lane0 = lax.iota(jnp.int32, L)
        part_v[:] = jnp.where(lane0 == 0, acc, 0.0)
        pltpu.sync_copy(part_v, shared.at[sid])
        plsc.subcore_barrier()

        @pl.when(sid == 0)
        def _():
            pltpu.sync_copy(shared, red_v)
            tot = jnp.zeros((L,), jnp.float32)
            for r in range(num_subcores):
                tot = tot + red_v[r, :]
            total = jnp.sum(tot)
            lane = lax.iota(jnp.int32, L)
            part_v[:] = jnp.where(lane == 0, total, 0.0)
            pltpu.sync_copy(part_v, out_hbm.at[cid])

    return triplet_loss_kernel


def kernel(eeg_embeddings, img_embeddings, target, triplets):
    info = plsc.get_sparse_core_info()
    num_cores, num_subcores = info.num_cores, info.num_subcores
    tri = triplets.astype(jnp.int32)
    aidx = tri[:, 0].reshape(T // 64, 64)
    pidx = tri[:, 1].reshape(T // 64, 64)
    nidx = tri[:, 2].reshape(T // 64, 64)
    fn = _build_kernel(num_cores, num_subcores)
    out = fn(eeg_embeddings, img_embeddings, aidx, pidx, nidx)
    loss = jnp.sum(out[:, 0]) * (1.0 / T)
    return (loss, jnp.asarray(T))
